# baseline (speedup 1.0000x reference)
import functools

import jax
import jax.numpy as jnp
from jax import lax
from jax.experimental import pallas as pl
from jax.experimental.pallas import tpu as pltpu
from jax.experimental.pallas import tpu_sc as plsc

N = 10000
E = 320000
D = 128
NC = 2
NS = 16
NW = NC * NS
K = 64
BLK_PER_W = 168
E_PAD = NW * BLK_PER_W * K
GROW = 10112
GARBAGE = N
ZROWS = GROW // NS
R = 400
G = N // R
BASE1 = 12000
OUT_ROWS = BASE1 + GROW
OFF1 = BASE1 // R

_mesh = plsc.VectorSubcoreMesh(core_axis_name="c", subcore_axis_name="s")


def _make_sc_agg():
  CHK = 24
  CP = BLK_PER_W // CHK
  U = 4
  UQ = CHK // U
  TOT = BLK_PER_W
  out_type = [jax.ShapeDtypeStruct((OUT_ROWS, D), jnp.float32)]
  scratch = [
      pltpu.VMEM((BLK_PER_W * K,), jnp.int32),
      pltpu.VMEM((2, CHK, K), jnp.int32),
      pltpu.VMEM((K, D), jnp.float32),
      pltpu.VMEM((K, D), jnp.float32),
      pltpu.VMEM((K, D), jnp.float32),
      pltpu.VMEM((K, D), jnp.float32),
      pltpu.VMEM_SHARED((GROW, D), jnp.float32),
      pltpu.SemaphoreType.DMA,
      pltpu.SemaphoreType.DMA,
      pltpu.SemaphoreType.DMA,
      pltpu.SemaphoreType.DMA,
      pltpu.SemaphoreType.DMA,
      pltpu.SemaphoreType.DMA,
      pltpu.SemaphoreType.DMA,
  ]

  def body(feats, srcf, dstb, zrows, aggp,
           sidx, didx, rows_a, rows_b, rows_c, rows_d, agg_sh,
           g0, g1, g2, g3, sem_s, sem_i0, sem_i1):
    cid = lax.axis_index("c")
    sid = lax.axis_index("s")
    wid = cid * NS + sid

    z0 = sid * ZROWS
    pltpu.sync_copy(zrows, agg_sh.at[pl.ds(z0, ZROWS)])

    blk0 = wid * BLK_PER_W
    pltpu.sync_copy(srcf.at[pl.ds(blk0 * K, BLK_PER_W * K)], sidx)
    sem_is = (sem_i0, sem_i1)
    pltpu.sync_copy(dstb.at[pl.ds(blk0, CHK)], didx.at[0])
    pltpu.async_copy(dstb.at[pl.ds(blk0 + CHK, CHK)], didx.at[1], sem_is[1])

    plsc.subcore_barrier()

    bufs = ((rows_a, g0), (rows_b, g1), (rows_c, g2), (rows_d, g3))

    def gather_start(j, buf, sem):
      pltpu.async_copy(feats.at[sidx.at[pl.ds(j * K, K)]], buf, sem)

    def gather_wait(buf, sem):
      pltpu.make_async_copy(feats.at[sidx.at[pl.ds(0, K)]], buf, sem).wait()

    def scatter_start(slab, lj, buf):
      pltpu.async_copy(buf, agg_sh.at[didx.at[slab, lj]], sem_s, add=True)

    def scatter_drain():
      pltpu.make_async_copy(rows_a, agg_sh.at[didx.at[0, 0]], sem_s).wait()

    def visit(slab, lj, b, drain, refill, j_next):
      if drain:
        scatter_drain()
      buf, gs = bufs[b]
      gather_wait(buf, gs)
      scatter_start(slab, lj, buf)
      if refill:
        tb, tgs = bufs[(b + 2) % U]
        gather_start(j_next, tb, tgs)

    gather_start(0, rows_a, g0)
    gather_start(1, rows_b, g1)

    for c in range(CP):
      slab = c % 2
      if c > 0:
        pltpu.make_async_copy(dstb.at[pl.ds(blk0, CHK)], didx.at[slab],
                              sem_is[slab]).wait()
      for b in range(U):
        j = c * CHK + b
        visit(slab, b, b, j >= 2, True, j + 2)
      if 1 <= c < CP - 1:
        pltpu.async_copy(dstb.at[pl.ds(blk0 + (c + 1) * CHK, CHK)],
                         didx.at[(c + 1) % 2], sem_is[(c + 1) % 2])

      if c < CP - 1:
        def grp(q, carry, c=c, slab=slab):
          for b in range(U):
            j = c * CHK + 4 * q + b
            visit(slab, 4 * q + b, b, True, True, j + 2)
          return carry
        lax.fori_loop(1, UQ, grp, 0)
      else:
        def grp(q, carry, c=c, slab=slab):
          for b in range(U):
            j = c * CHK + 4 * q + b
            visit(slab, 4 * q + b, b, True, True, j + 2)
          return carry
        lax.fori_loop(1, UQ - 1, grp, 0)
        for b in range(U):
          j = c * CHK + (UQ - 1) * U + b
          visit(slab, (UQ - 1) * U + b, b, True, j + 2 < TOT, j + 2)

    scatter_drain()
    scatter_drain()

    plsc.subcore_barrier()

    pltpu.sync_copy(agg_sh.at[pl.ds(z0, ZROWS)],
                    aggp.at[pl.ds(cid * BASE1 + z0, ZROWS)])

  return pl.kernel(body, out_type=out_type, mesh=_mesh, scratch_types=scratch)


_sc_agg = _make_sc_agg()


def _make_sc_cnt():
  out_type = [jax.ShapeDtypeStruct((OUT_ROWS, D), jnp.float32)]
  scratch = [
      pltpu.VMEM((BLK_PER_W, K), jnp.int32),
      pltpu.VMEM((K, D), jnp.float32),
      pltpu.VMEM_SHARED((GROW, D), jnp.float32),
      pltpu.SemaphoreType.DMA,
  ]

  def body(dstb, zrows, ones_h, cntp, didx, ones2d, cnt_sh, sem):
    cid = lax.axis_index("c")
    sid = lax.axis_index("s")
    wid = cid * NS + sid

    z0 = sid * ZROWS
    pltpu.sync_copy(zrows, cnt_sh.at[pl.ds(z0, ZROWS)])
    pltpu.sync_copy(ones_h, ones2d)

    pltpu.sync_copy(dstb.at[pl.ds(wid * BLK_PER_W, BLK_PER_W)], didx)
    plsc.subcore_barrier()

    def loop_body(j, carry):
      pltpu.async_copy(ones2d, cnt_sh.at[didx.at[j]], sem, add=True)
      return carry

    lax.fori_loop(0, BLK_PER_W, loop_body, 0)

    def drain_body(j, carry):
      pltpu.make_async_copy(ones2d, cnt_sh.at[didx.at[0]], sem).wait()
      return carry

    lax.fori_loop(0, BLK_PER_W, drain_body, 0)

    plsc.subcore_barrier()
    pltpu.sync_copy(cnt_sh.at[pl.ds(z0, ZROWS)],
                    cntp.at[pl.ds(cid * BASE1 + z0, ZROWS)])

  return pl.kernel(body, out_type=out_type, mesh=_mesh, scratch_types=scratch)


_sc_cnt = _make_sc_cnt()


def _make_tc_stage(relu: bool):
  dn = (((1,), (1,)), ((), ()))

  def body(p0, p1, c0, c1, x, wl, b, wr, o):
    cnt = jnp.maximum(c0[:, :1] + c1[:, :1], 1.0)
    agg = (p0[...] + p1[...]) * (1.0 / cnt)
    y = lax.dot_general(agg, wl[...], dn, preferred_element_type=jnp.float32)
    y = y + b[...]
    y = y + lax.dot_general(x[...], wr[...], dn,
                            preferred_element_type=jnp.float32)
    if relu:
      y = jnp.maximum(y, 0.0)
    o[...] = y

  return pl.pallas_call(
      body,
      grid=(G,),
      in_specs=[
          pl.BlockSpec((R, D), lambda i: (i, 0)),
          pl.BlockSpec((R, D), lambda i: (i + OFF1, 0)),
          pl.BlockSpec((R, D), lambda i: (i, 0)),
          pl.BlockSpec((R, D), lambda i: (i + OFF1, 0)),
          pl.BlockSpec((R, D), lambda i: (i, 0)),
          pl.BlockSpec((D, D), lambda i: (0, 0)),
          pl.BlockSpec((1, D), lambda i: (0, 0)),
          pl.BlockSpec((D, D), lambda i: (0, 0)),
      ],
      out_specs=pl.BlockSpec((R, D), lambda i: (i, 0)),
      out_shape=jax.ShapeDtypeStruct((N, D), jnp.float32),
  )


_tc_relu = _make_tc_stage(True)
_tc_lin = _make_tc_stage(False)


def kernel(x, edge_index, W1_l, b1_l, W1_r, W2_l, b2_l, W2_r):
  src = edge_index[0].astype(jnp.int32)
  dst = edge_index[1].astype(jnp.int32)
  pad = E_PAD - E
  pad_src = (jnp.arange(pad, dtype=jnp.int32) * 163) % N
  pad_dst = GARBAGE + jnp.arange(pad, dtype=jnp.int32) % (GROW - N)
  srcf = jnp.concatenate([src, pad_src])
  dstb = jnp.concatenate([dst, pad_dst]).reshape(-1, K)
  zrows = jnp.zeros((ZROWS, D), jnp.float32)
  ones_h = jnp.ones((K, D), jnp.float32)

  (cnt2,) = _sc_cnt(dstb, zrows, ones_h)

  (aggp,) = _sc_agg(x, srcf, dstb, zrows)
  h = _tc_relu(aggp, aggp, cnt2, cnt2, x, W1_l, b1_l.reshape(1, D), W1_r)

  (aggp2,) = _sc_agg(h, srcf, dstb, zrows)
  out = _tc_lin(aggp2, aggp2, cnt2, cnt2, h, W2_l, b2_l.reshape(1, D), W2_r)
  return out


# speedup vs baseline: 1.1220x; 1.1220x over previous
import functools

import jax
import jax.numpy as jnp
from jax import lax
from jax.experimental import pallas as pl
from jax.experimental.pallas import tpu as pltpu
from jax.experimental.pallas import tpu_sc as plsc

N = 10000
E = 320000
D = 128
NC = 2
NS = 16
NW = NC * NS
K = 64
BLK_PER_W = 168
E_PAD = NW * BLK_PER_W * K
GROW = 10112
GARBAGE = N
ZROWS = GROW // NS

_mesh = plsc.VectorSubcoreMesh(core_axis_name="c", subcore_axis_name="s")


def _make_sc_agg():
  CHK = 24
  CP = BLK_PER_W // CHK
  out_type = [jax.ShapeDtypeStruct((NC * GROW, D), jnp.float32)]
  scratch = [
      pltpu.VMEM((BLK_PER_W * K,), jnp.int32),
      pltpu.VMEM((2, CHK, K), jnp.int32),
      pltpu.VMEM((K, D), jnp.float32),
      pltpu.VMEM((K, D), jnp.float32),
      pltpu.VMEM((K, D), jnp.float32),
      pltpu.VMEM_SHARED((GROW, D), jnp.float32),
      pltpu.SemaphoreType.DMA,
      pltpu.SemaphoreType.DMA,
      pltpu.SemaphoreType.DMA,
      pltpu.SemaphoreType.DMA,
      pltpu.SemaphoreType.DMA,
  ]

  def body(feats, srcf, dstb, zagg, aggp,
           sidx, didx, rows_a, rows_b, rows_c, agg_sh,
           sem_a, sem_b, sem_c, sem_i0, sem_i1):
    cid = lax.axis_index("c")
    sid = lax.axis_index("s")
    wid = cid * NS + sid

    z0 = sid * ZROWS
    pltpu.sync_copy(zagg.at[pl.ds(z0, ZROWS)], agg_sh.at[pl.ds(z0, ZROWS)])

    blk0 = wid * BLK_PER_W
    pltpu.sync_copy(srcf.at[pl.ds(blk0 * K, BLK_PER_W * K)], sidx)
    sem_is = (sem_i0, sem_i1)
    pltpu.sync_copy(dstb.at[pl.ds(blk0, CHK)], didx.at[0])
    pltpu.async_copy(dstb.at[pl.ds(blk0 + CHK, CHK)], didx.at[1], sem_is[1])

    plsc.subcore_barrier()

    def gather_start(j, buf, sem):
      pltpu.async_copy(feats.at[sidx.at[pl.ds(j * K, K)]], buf, sem)

    def gather_wait(buf, sem):
      pltpu.make_async_copy(feats.at[sidx.at[pl.ds(0, K)]], buf, sem).wait()

    def scatter(slab, lj, buf):
      pltpu.sync_copy(buf, agg_sh.at[didx.at[slab, lj]], add=True)

    bufs = ((rows_a, sem_a), (rows_b, sem_b), (rows_c, sem_c))
    for b, (buf, sem) in enumerate(bufs):
      gather_start(b, buf, sem)

    for c in range(CP):
      slab = c % 2
      if c > 0:
        pltpu.make_async_copy(dstb.at[pl.ds(blk0, CHK)], didx.at[slab],
                              sem_is[slab]).wait()
      last = c == CP - 1
      iters = CHK // 3 - (1 if last else 0)

      def chunk_body(q, carry, c=c, slab=slab, last=last):
        j = c * CHK + 3 * q
        for b, (buf, sem) in enumerate(bufs):
          gather_wait(buf, sem)
          scatter(slab, 3 * q + b, buf)
          if not last:
            gather_start(j + b + 3, buf, sem)
          else:
            gather_start(jnp.minimum(j + b + 3, BLK_PER_W - 1), buf, sem)
        return carry

      lax.fori_loop(0, iters, chunk_body, 0)
      if last:
        for b, (buf, sem) in enumerate(bufs):
          gather_wait(buf, sem)
          scatter(slab, CHK - 3 + b, buf)
      elif c + 2 < CP:
        pltpu.async_copy(dstb.at[pl.ds(blk0 + (c + 2) * CHK, CHK)],
                         didx.at[slab], sem_is[slab])

    plsc.subcore_barrier()

    pltpu.sync_copy(agg_sh.at[pl.ds(z0, ZROWS)],
                    aggp.at[pl.ds(cid * GROW + z0, ZROWS)])

  return pl.kernel(body, out_type=out_type, mesh=_mesh, scratch_types=scratch)


_sc_agg = _make_sc_agg()


def _make_sc_cnt():
  out_type = [jax.ShapeDtypeStruct((NC * GROW, D), jnp.float32)]
  scratch = [
      pltpu.VMEM((BLK_PER_W, K), jnp.int32),
      pltpu.VMEM((K, D), jnp.float32),
      pltpu.VMEM_SHARED((GROW, D), jnp.float32),
      pltpu.SemaphoreType.DMA,
  ]

  def body(dstb, zcnt, ones_h, cntp, didx, ones2d, cnt_sh, sem):
    cid = lax.axis_index("c")
    sid = lax.axis_index("s")
    wid = cid * NS + sid

    z0 = sid * ZROWS
    pltpu.sync_copy(zcnt.at[pl.ds(z0, ZROWS)], cnt_sh.at[pl.ds(z0, ZROWS)])
    pltpu.sync_copy(ones_h, ones2d)

    pltpu.sync_copy(dstb.at[pl.ds(wid * BLK_PER_W, BLK_PER_W)], didx)
    plsc.subcore_barrier()

    def loop_body(j, carry):
      pltpu.sync_copy(ones2d, cnt_sh.at[didx.at[j]], add=True)
      return carry

    lax.fori_loop(0, BLK_PER_W, loop_body, 0)

    plsc.subcore_barrier()
    pltpu.sync_copy(cnt_sh.at[pl.ds(z0, ZROWS)],
                    cntp.at[pl.ds(cid * GROW + z0, ZROWS)])

  return pl.kernel(body, out_type=out_type, mesh=_mesh, scratch_types=scratch)


_sc_cnt = _make_sc_cnt()


def _make_tc_stage(relu: bool):
  R = 1000
  G = N // R
  dn = (((1,), (1,)), ((), ()))

  def body(p0, p1, c0, c1, x, wl, b, wr, o):
    cnt = jnp.maximum(c0[...] + c1[...], 1.0)
    agg = (p0[...] + p1[...]) * (1.0 / cnt)
    y = lax.dot_general(agg, wl[...], dn, preferred_element_type=jnp.float32)
    y = y + b[...]
    y = y + lax.dot_general(x[...], wr[...], dn,
                            preferred_element_type=jnp.float32)
    if relu:
      y = jnp.maximum(y, 0.0)
    o[...] = y

  return pl.pallas_call(
      body,
      grid=(G,),
      in_specs=[
          pl.BlockSpec((R, D), lambda i: (i, 0)),
          pl.BlockSpec((R, D), lambda i: (i, 0)),
          pl.BlockSpec((R, 1), lambda i: (i, 0)),
          pl.BlockSpec((R, 1), lambda i: (i, 0)),
          pl.BlockSpec((R, D), lambda i: (i, 0)),
          pl.BlockSpec((D, D), lambda i: (0, 0)),
          pl.BlockSpec((1, D), lambda i: (0, 0)),
          pl.BlockSpec((D, D), lambda i: (0, 0)),
      ],
      out_specs=pl.BlockSpec((R, D), lambda i: (i, 0)),
      out_shape=jax.ShapeDtypeStruct((N, D), jnp.float32),
  )


_tc_relu = _make_tc_stage(True)
_tc_lin = _make_tc_stage(False)


def kernel(x, edge_index, W1_l, b1_l, W1_r, W2_l, b2_l, W2_r):
  src = edge_index[0].astype(jnp.int32)
  dst = edge_index[1].astype(jnp.int32)
  pad = E_PAD - E
  pad_src = (jnp.arange(pad, dtype=jnp.int32) * 163) % N
  pad_dst = GARBAGE + jnp.arange(pad, dtype=jnp.int32) % (GROW - N)
  srcf = jnp.concatenate([src, pad_src])
  dstb = jnp.concatenate([dst, pad_dst]).reshape(-1, K)
  zagg = jnp.zeros((GROW, D), jnp.float32)
  ones_h = jnp.ones((K, D), jnp.float32)

  (cnt16,) = _sc_cnt(dstb, zagg, ones_h)
  c0 = cnt16[:N, :1]
  c1 = cnt16[GROW:GROW + N, :1]

  (aggp,) = _sc_agg(x, srcf, dstb, zagg)
  h = _tc_relu(aggp[:N], aggp[GROW:GROW + N], c0, c1, x,
               W1_l, b1_l.reshape(1, D), W1_r)

  (aggp2,) = _sc_agg(h, srcf, dstb, zagg)
  out = _tc_lin(aggp2[:N], aggp2[GROW:GROW + N], c0, c1, h,
                W2_l, b2_l.reshape(1, D), W2_r)
  return out
